# baseline (speedup 1.0000x reference)
import functools

import jax
import jax.numpy as jnp
from jax import lax
from jax.experimental import pallas as pl
from jax.experimental.pallas import tpu as pltpu
from jax.experimental.pallas import tpu_sc as plsc

_NC = 2
_NS = 16
_NW = _NC * _NS
_LANES = 16

_F32 = jnp.float32
_CH = 80


def _zero16():
    return jnp.zeros((_LANES,), _F32)


def _zero_acc(z_hbm, acc_sh, s, rpt, tail):
    pltpu.sync_copy(z_hbm.at[pl.ds(s * rpt, rpt)],
                    acc_sh.at[pl.ds(s * rpt, rpt)])
    if tail:
        @pl.when(s == 0)
        def _():
            pltpu.sync_copy(z_hbm.at[pl.ds(_NS * rpt, tail)],
                            acc_sh.at[pl.ds(_NS * rpt, tail)])


def _read_out(acc_sh, out, c, s, rpt, tail):
    pltpu.sync_copy(acc_sh.at[pl.ds(s * rpt, rpt)],
                    out.at[c, pl.ds(s * rpt, rpt)])
    if tail:
        @pl.when(s == 0)
        def _():
            pltpu.sync_copy(acc_sh.at[pl.ds(_NS * rpt, tail)],
                            out.at[c, pl.ds(_NS * rpt, tail)])


def _edge_agg_body(N, D, E, ea_hbm, dst_hbm, w0b_hbm, z_hbm, out_ea, out_s0,
                   idxa, idxb, rowsa, rowsb, acc_sh,
                   sia, sib, sga, sgb, ssa, ssb):
    c = lax.axis_index("c")
    s = lax.axis_index("s")
    wid = s * _NC + c
    ept = E // _NW
    nch = ept // _CH
    npairs = nch // 2
    rpt = (N // _NS) & ~7
    tail = N - rpt * _NS

    def idx_start(i, idxv, sem):
        base = wid * ept + i * _CH
        pltpu.async_copy(dst_hbm.at[pl.ds(base, _CH)], idxv, sem)

    def idx_wait(idxv, sem):
        pltpu.make_async_copy(dst_hbm.at[pl.ds(0, _CH)], idxv, sem).wait()

    def fetch_start(i, rowsv, sem):
        base = wid * ept + i * _CH
        pltpu.async_copy(ea_hbm.at[pl.ds(base, _CH)], rowsv, sem)

    def rows_wait(rowsv, sem):
        pltpu.make_async_copy(ea_hbm.at[pl.ds(0, _CH)], rowsv, sem).wait()

    def scat_start(idxv, rowsv, sem):
        pltpu.async_copy(rowsv, acc_sh.at[idxv], sem, add=True)

    _zero_acc(z_hbm, acc_sh, s, rpt, tail)
    plsc.subcore_barrier()

    idx_start(0, idxa, sia)
    idx_wait(idxa, sia)
    fetch_start(0, rowsa, sga)
    idx_start(1, idxb, sib)

    def pair1(j, _):
        i = 2 * j
        rows_wait(rowsa, sga)
        scat_start(idxa, rowsa, ssa)
        idx_wait(idxb, sib)
        fetch_start(i + 1, rowsb, sgb)
        rows_wait(rowsa, ssa)
        idx_start(i + 2, idxa, sia)
        rows_wait(rowsb, sgb)
        scat_start(idxb, rowsb, ssb)
        idx_wait(idxa, sia)
        fetch_start(i + 2, rowsa, sga)
        rows_wait(rowsb, ssb)

        @pl.when(i + 3 < nch)
        def _():
            idx_start(i + 3, idxb, sib)
        return 0
    lax.fori_loop(0, npairs, pair1, 0)
    rows_wait(rowsa, sga)
    scat_start(idxa, rowsa, ssa)
    rows_wait(rowsa, ssa)
    plsc.subcore_barrier()

    _read_out(acc_sh, out_ea, c, s, rpt, tail)
    _zero_acc(z_hbm, acc_sh, s, rpt, tail)
    plsc.subcore_barrier()

    pltpu.sync_copy(w0b_hbm, rowsa)
    idx_start(0, idxa, sia)
    idx_start(1, idxb, sib)

    def pair2(j, _):
        i = 2 * j
        idx_wait(idxa, sia)
        scat_start(idxa, rowsa, ssa)
        idx_wait(idxb, sib)
        scat_start(idxb, rowsa, ssb)
        rows_wait(rowsa, ssa)
        idx_start(i + 2, idxa, sia)
        rows_wait(rowsa, ssb)

        @pl.when(i + 3 < nch)
        def _():
            idx_start(i + 3, idxb, sib)
        return 0
    lax.fori_loop(0, npairs, pair2, 0)
    idx_wait(idxa, sia)
    scat_start(idxa, rowsa, ssa)
    rows_wait(rowsa, ssa)
    plsc.subcore_barrier()

    _read_out(acc_sh, out_s0, c, s, rpt, tail)


def _edge_agg_call(edge_attr, dst, w0b, zeros_nd, N):
    E, D = edge_attr.shape
    nch = (E // _NW) // _CH
    assert E % (_NW * _CH) == 0 and N % _NS == 0 and nch % 2 == 1 and nch >= 3
    mesh = plsc.VectorSubcoreMesh(core_axis_name="c", subcore_axis_name="s")
    f = pl.kernel(
        functools.partial(_edge_agg_body, N, D, E),
        out_type=(
            jax.ShapeDtypeStruct((_NC, N, D), _F32),
            jax.ShapeDtypeStruct((_NC, N, D), _F32),
        ),
        mesh=mesh,
        scratch_types=[
            pltpu.VMEM((_CH,), jnp.int32),
            pltpu.VMEM((_CH,), jnp.int32),
            pltpu.VMEM((_CH, D), _F32),
            pltpu.VMEM((_CH, D), _F32),
            pltpu.VMEM_SHARED((N, D), _F32),
            pltpu.SemaphoreType.DMA,
            pltpu.SemaphoreType.DMA,
            pltpu.SemaphoreType.DMA,
            pltpu.SemaphoreType.DMA,
            pltpu.SemaphoreType.DMA,
            pltpu.SemaphoreType.DMA,
        ],
    )
    return f(edge_attr, dst, w0b, zeros_nd)


def _spmv_body(N, D, E, hw_hbm, src_hbm, dst_hbm, z_hbm, out_s,
               sidxa, didxa, sidxb, didxb, rowsa, rowsb, acc_sh,
               sia, sib, sga, sgb, ssa, ssb):
    c = lax.axis_index("c")
    s = lax.axis_index("s")
    wid = s * _NC + c
    ept = E // _NW
    nch = ept // _CH
    npairs = nch // 2
    rpt = (N // _NS) & ~7
    tail = N - rpt * _NS

    def idx_start(i, sidxv, didxv, sem):
        base = wid * ept + i * _CH
        pltpu.async_copy(src_hbm.at[pl.ds(base, _CH)], sidxv, sem)
        pltpu.async_copy(dst_hbm.at[pl.ds(base, _CH)], didxv, sem)

    def idx_wait(sidxv, didxv, sem):
        pltpu.make_async_copy(src_hbm.at[pl.ds(0, _CH)], sidxv, sem).wait()
        pltpu.make_async_copy(dst_hbm.at[pl.ds(0, _CH)], didxv, sem).wait()

    def gather_start(sidxv, rowsv, sem):
        pltpu.async_copy(hw_hbm.at[sidxv], rowsv, sem)

    def rows_wait(rowsv, sem):
        pltpu.make_async_copy(hw_hbm.at[pl.ds(0, _CH)], rowsv, sem).wait()

    def scat_start(didxv, rowsv, sem):
        pltpu.async_copy(rowsv, acc_sh.at[didxv], sem, add=True)

    _zero_acc(z_hbm, acc_sh, s, rpt, tail)
    plsc.subcore_barrier()

    idx_start(0, sidxa, didxa, sia)
    idx_wait(sidxa, didxa, sia)
    gather_start(sidxa, rowsa, sga)
    idx_start(1, sidxb, didxb, sib)

    def pair(j, _):
        i = 2 * j
        rows_wait(rowsa, sga)
        scat_start(didxa, rowsa, ssa)
        idx_wait(sidxb, didxb, sib)
        gather_start(sidxb, rowsb, sgb)
        rows_wait(rowsa, ssa)
        idx_start(i + 2, sidxa, didxa, sia)
        rows_wait(rowsb, sgb)
        scat_start(didxb, rowsb, ssb)
        idx_wait(sidxa, didxa, sia)
        gather_start(sidxa, rowsa, sga)
        rows_wait(rowsb, ssb)

        @pl.when(i + 3 < nch)
        def _():
            idx_start(i + 3, sidxb, didxb, sib)
        return 0
    lax.fori_loop(0, npairs, pair, 0)
    rows_wait(rowsa, sga)
    scat_start(didxa, rowsa, ssa)
    rows_wait(rowsa, ssa)
    plsc.subcore_barrier()

    _read_out(acc_sh, out_s, c, s, rpt, tail)


def _spmv(hw, src, dst, zeros_nd):
    N, D = hw.shape
    E = src.shape[0]
    nch = (E // _NW) // _CH
    assert E % (_NW * _CH) == 0 and N % _NS == 0 and nch % 2 == 1 and nch >= 3
    mesh = plsc.VectorSubcoreMesh(core_axis_name="c", subcore_axis_name="s")
    f = pl.kernel(
        functools.partial(_spmv_body, N, D, E),
        out_type=jax.ShapeDtypeStruct((_NC, N, D), _F32),
        mesh=mesh,
        scratch_types=[
            pltpu.VMEM((_CH,), jnp.int32),
            pltpu.VMEM((_CH,), jnp.int32),
            pltpu.VMEM((_CH,), jnp.int32),
            pltpu.VMEM((_CH,), jnp.int32),
            pltpu.VMEM((_CH, D), _F32),
            pltpu.VMEM((_CH, D), _F32),
            pltpu.VMEM_SHARED((N, D), _F32),
            pltpu.SemaphoreType.DMA,
            pltpu.SemaphoreType.DMA,
            pltpu.SemaphoreType.DMA,
            pltpu.SemaphoreType.DMA,
            pltpu.SemaphoreType.DMA,
            pltpu.SemaphoreType.DMA,
        ],
    )
    return f(hw, src, dst, zeros_nd)


_BN = 1000


def _tc1_body(nt_ref, w0_ref, b0_ref, w1_ref, b1_ref, ea_ref, s0_ref, out_ref):
    w0 = (jnp.dot(nt_ref[...], w0_ref[...], preferred_element_type=_F32)
          + b0_ref[...])
    h1 = jnp.maximum(
        s0_ref[0] + s0_ref[1] + ea_ref[0] + ea_ref[1] + w0, 0.0)
    out_ref[...] = (jnp.dot(h1, w1_ref[...], preferred_element_type=_F32)
                    + b1_ref[...])


def _tc1(node_table, W0, b0, W1, b1, ea_part, s0_part):
    _, N, D = ea_part.shape
    grid = (N // _BN,)
    return pl.pallas_call(
        _tc1_body,
        grid=grid,
        in_specs=[
            pl.BlockSpec((1, D), lambda i: (0, 0)),
            pl.BlockSpec((D, D), lambda i: (0, 0)),
            pl.BlockSpec((1, D), lambda i: (0, 0)),
            pl.BlockSpec((D, D), lambda i: (0, 0)),
            pl.BlockSpec((1, D), lambda i: (0, 0)),
            pl.BlockSpec((_NC, _BN, D), lambda i: (0, i, 0)),
            pl.BlockSpec((_NC, _BN, D), lambda i: (0, i, 0)),
        ],
        out_specs=pl.BlockSpec((_BN, D), lambda i: (i, 0)),
        out_shape=jax.ShapeDtypeStruct((N, D), _F32),
    )(node_table, W0, b0, W1, b1, ea_part, s0_part)


def _tc_mid_body(s_ref, ea_ref, hw_ref, w_ref, b_ref, out_ref):
    h = jnp.maximum(s_ref[0] + s_ref[1] + ea_ref[0] + ea_ref[1] + hw_ref[...],
                    0.0)
    out_ref[...] = (jnp.dot(h, w_ref[...], preferred_element_type=_F32)
                    + b_ref[...])


def _tc_mid(s_part, ea_part, hw, W, b):
    N, D = hw.shape
    grid = (N // _BN,)
    return pl.pallas_call(
        _tc_mid_body,
        grid=grid,
        in_specs=[
            pl.BlockSpec((_NC, _BN, D), lambda i: (0, i, 0)),
            pl.BlockSpec((_NC, _BN, D), lambda i: (0, i, 0)),
            pl.BlockSpec((_BN, D), lambda i: (i, 0)),
            pl.BlockSpec((D, D), lambda i: (0, 0)),
            pl.BlockSpec((1, D), lambda i: (0, 0)),
        ],
        out_specs=pl.BlockSpec((_BN, D), lambda i: (i, 0)),
        out_shape=jax.ShapeDtypeStruct((N, D), _F32),
    )(s_part, ea_part, hw, W, b)


def _tc_final_body(G, C, nblocks, s_ref, ea_ref, hw_ref, batch_ref, pw_ref,
                   pb_ref, out_ref, sums_acc, cnt_acc):
    i = pl.program_id(0)

    @pl.when(i == 0)
    def _():
        sums_acc[...] = jnp.zeros_like(sums_acc)
        cnt_acc[...] = jnp.zeros_like(cnt_acc)

    h3 = s_ref[0] + s_ref[1] + ea_ref[0] + ea_ref[1] + hw_ref[...]
    b = batch_ref[...]
    onehot = (b == lax.broadcasted_iota(jnp.int32, (_BN, G), 1)).astype(_F32)
    dn = (((0,), (0,)), ((), ()))
    sums_acc[...] += lax.dot_general(onehot, h3, dn,
                                     preferred_element_type=_F32)
    cnt_acc[...] += lax.dot_general(onehot, jnp.ones_like(h3), dn,
                                    preferred_element_type=_F32)

    @pl.when(i == nblocks - 1)
    def _():
        pooled = sums_acc[...] / jnp.maximum(cnt_acc[...], 1.0)
        out_ref[...] = (jnp.dot(pooled, pw_ref[...],
                                preferred_element_type=_F32) + pb_ref[...])


def _tc_final(s_part, ea_part, hw, batch_col, perc_W, perc_b, G):
    N, D = hw.shape
    C = perc_W.shape[1]
    nblocks = N // _BN
    return pl.pallas_call(
        functools.partial(_tc_final_body, G, C, nblocks),
        grid=(nblocks,),
        in_specs=[
            pl.BlockSpec((_NC, _BN, D), lambda i: (0, i, 0)),
            pl.BlockSpec((_NC, _BN, D), lambda i: (0, i, 0)),
            pl.BlockSpec((_BN, D), lambda i: (i, 0)),
            pl.BlockSpec((_BN, 1), lambda i: (i, 0)),
            pl.BlockSpec((D, C), lambda i: (0, 0)),
            pl.BlockSpec((1, C), lambda i: (0, 0)),
        ],
        out_specs=pl.BlockSpec((G, C), lambda i: (0, 0)),
        out_shape=jax.ShapeDtypeStruct((G, C), _F32),
        scratch_shapes=[
            pltpu.VMEM((G, D), _F32),
            pltpu.VMEM((G, D), _F32),
        ],
    )(s_part, ea_part, hw, batch_col, perc_W, perc_b)


def kernel(x, edge_index, edge_attr, batch, node_table, layer_W, layer_b,
           perc_W, perc_b):
    N = x.shape[0]
    E, D = edge_attr.shape
    L = layer_W.shape[0]
    C = perc_W.shape[1]
    G = 128

    batch_col = batch.reshape(N, 1)
    b_rows = layer_b.reshape(L, 1, D)
    pb_row = perc_b.reshape(1, C)

    w0 = node_table[0] @ layer_W[0] + layer_b[0]
    w0b = jnp.broadcast_to(w0.reshape(1, D), (_CH, D))
    zeros_nd = jnp.zeros((N, D), _F32)

    src = edge_index[0]
    dst = edge_index[1]

    ea_part, s0_part = _edge_agg_call(edge_attr, dst, w0b, zeros_nd, N)

    hw = _tc1(node_table, layer_W[0], b_rows[0], layer_W[1], b_rows[1],
              ea_part, s0_part)

    for l in range(2, L):
        s_part = _spmv(hw, src, dst, zeros_nd)
        hw = _tc_mid(s_part, ea_part, hw, layer_W[l], b_rows[l])

    s_part = _spmv(hw, src, dst, zeros_nd)
    return _tc_final(s_part, ea_part, hw, batch_col, perc_W, pb_row, G)


# speedup vs baseline: 1.0203x; 1.0203x over previous
import functools

import jax
import jax.numpy as jnp
from jax import lax
from jax.experimental import pallas as pl
from jax.experimental.pallas import tpu as pltpu
from jax.experimental.pallas import tpu_sc as plsc

_NC = 2
_NS = 16
_NW = _NC * _NS
_LANES = 16

_F32 = jnp.float32
_CH = 80


def _zero16():
    return jnp.zeros((_LANES,), _F32)


def _zero_acc(z_hbm, acc_sh, s, rpt, tail):
    pltpu.sync_copy(z_hbm.at[pl.ds(s * rpt, rpt)],
                    acc_sh.at[pl.ds(s * rpt, rpt)])
    if tail:
        @pl.when(s == 0)
        def _():
            pltpu.sync_copy(z_hbm.at[pl.ds(_NS * rpt, tail)],
                            acc_sh.at[pl.ds(_NS * rpt, tail)])


def _read_out(acc_sh, out, c, s, rpt, tail):
    pltpu.sync_copy(acc_sh.at[pl.ds(s * rpt, rpt)],
                    out.at[c, pl.ds(s * rpt, rpt)])
    if tail:
        @pl.when(s == 0)
        def _():
            pltpu.sync_copy(acc_sh.at[pl.ds(_NS * rpt, tail)],
                            out.at[c, pl.ds(_NS * rpt, tail)])


def _edge_agg_body(N, D, E, ea_hbm, dst_hbm, w0b_hbm, z_hbm, out_ea, out_s0,
                   idxa, idxb, rowsa, rowsb, acc_sh,
                   sia, sib, sga, sgb, ssa, ssb):
    c = lax.axis_index("c")
    s = lax.axis_index("s")
    wid = s * _NC + c
    ept = E // _NW
    nch = ept // _CH
    npairs = nch // 2
    rpt = (N // _NS) & ~7
    tail = N - rpt * _NS

    def idx_start(i, idxv, sem):
        base = wid * ept + i * _CH
        pltpu.async_copy(dst_hbm.at[pl.ds(base, _CH)], idxv, sem)

    def idx_wait(idxv, sem):
        pltpu.make_async_copy(dst_hbm.at[pl.ds(0, _CH)], idxv, sem).wait()

    def fetch_start(i, rowsv, sem):
        base = wid * ept + i * _CH
        pltpu.async_copy(ea_hbm.at[pl.ds(base, _CH)], rowsv, sem)

    def rows_wait(rowsv, sem):
        pltpu.make_async_copy(ea_hbm.at[pl.ds(0, _CH)], rowsv, sem).wait()

    def scat_start(idxv, rowsv, sem):
        pltpu.async_copy(rowsv, acc_sh.at[idxv], sem, add=True)

    _zero_acc(z_hbm, acc_sh, s, rpt, tail)
    plsc.subcore_barrier()

    idx_start(0, idxa, sia)
    idx_wait(idxa, sia)
    fetch_start(0, rowsa, sga)
    idx_start(1, idxb, sib)

    def pair1(j, _):
        i = 2 * j
        rows_wait(rowsa, sga)
        scat_start(idxa, rowsa, ssa)
        idx_wait(idxb, sib)
        fetch_start(i + 1, rowsb, sgb)
        rows_wait(rowsa, ssa)
        idx_start(i + 2, idxa, sia)
        rows_wait(rowsb, sgb)
        scat_start(idxb, rowsb, ssb)
        idx_wait(idxa, sia)
        fetch_start(i + 2, rowsa, sga)
        rows_wait(rowsb, ssb)

        @pl.when(i + 3 < nch)
        def _():
            idx_start(i + 3, idxb, sib)
        return 0
    lax.fori_loop(0, npairs, pair1, 0)
    rows_wait(rowsa, sga)
    scat_start(idxa, rowsa, ssa)
    rows_wait(rowsa, ssa)
    plsc.subcore_barrier()

    _read_out(acc_sh, out_ea, c, s, rpt, tail)
    _zero_acc(z_hbm, acc_sh, s, rpt, tail)
    plsc.subcore_barrier()

    pltpu.sync_copy(w0b_hbm, rowsa)
    idx_start(0, idxa, sia)
    idx_start(1, idxb, sib)

    def pair2(j, _):
        i = 2 * j
        idx_wait(idxa, sia)
        scat_start(idxa, rowsa, ssa)
        idx_wait(idxb, sib)
        scat_start(idxb, rowsa, ssb)
        rows_wait(rowsa, ssa)
        idx_start(i + 2, idxa, sia)
        rows_wait(rowsa, ssb)

        @pl.when(i + 3 < nch)
        def _():
            idx_start(i + 3, idxb, sib)
        return 0
    lax.fori_loop(0, npairs, pair2, 0)
    idx_wait(idxa, sia)
    scat_start(idxa, rowsa, ssa)
    rows_wait(rowsa, ssa)
    plsc.subcore_barrier()

    _read_out(acc_sh, out_s0, c, s, rpt, tail)


def _edge_agg_call(edge_attr, dst, w0b, zeros_nd, N):
    E, D = edge_attr.shape
    nch = (E // _NW) // _CH
    assert E % (_NW * _CH) == 0 and N % _NS == 0 and nch % 2 == 1 and nch >= 3
    mesh = plsc.VectorSubcoreMesh(core_axis_name="c", subcore_axis_name="s")
    f = pl.kernel(
        functools.partial(_edge_agg_body, N, D, E),
        out_type=(
            jax.ShapeDtypeStruct((_NC, N, D), _F32),
            jax.ShapeDtypeStruct((_NC, N, D), _F32),
        ),
        mesh=mesh,
        scratch_types=[
            pltpu.VMEM((_CH,), jnp.int32),
            pltpu.VMEM((_CH,), jnp.int32),
            pltpu.VMEM((_CH, D), _F32),
            pltpu.VMEM((_CH, D), _F32),
            pltpu.VMEM_SHARED((N, D), _F32),
            pltpu.SemaphoreType.DMA,
            pltpu.SemaphoreType.DMA,
            pltpu.SemaphoreType.DMA,
            pltpu.SemaphoreType.DMA,
            pltpu.SemaphoreType.DMA,
            pltpu.SemaphoreType.DMA,
        ],
    )
    return f(edge_attr, dst, w0b, zeros_nd)


def _spmv_body(N, D, E, hw_hbm, src_hbm, dst_hbm, z_hbm, out_s,
               sidxa, didxa, sidxb, didxb, rowsa, rowsb, acc_sh,
               sia, sib, sga, sgb, ssa, ssb):
    c = lax.axis_index("c")
    s = lax.axis_index("s")
    wid = s * _NC + c
    ept = E // _NW
    nch = ept // _CH
    npairs = nch // 2
    rpt = (N // _NS) & ~7
    tail = N - rpt * _NS

    def idx_start(i, sidxv, didxv, sem):
        base = wid * ept + i * _CH
        pltpu.async_copy(src_hbm.at[pl.ds(base, _CH)], sidxv, sem)
        pltpu.async_copy(dst_hbm.at[pl.ds(base, _CH)], didxv, sem)

    def idx_wait(sidxv, didxv, sem):
        pltpu.make_async_copy(src_hbm.at[pl.ds(0, _CH)], sidxv, sem).wait()
        pltpu.make_async_copy(dst_hbm.at[pl.ds(0, _CH)], didxv, sem).wait()

    def gather_start(sidxv, rowsv, sem):
        pltpu.async_copy(hw_hbm.at[sidxv], rowsv, sem)

    def rows_wait(rowsv, sem):
        pltpu.make_async_copy(hw_hbm.at[pl.ds(0, _CH)], rowsv, sem).wait()

    def scat_start(didxv, rowsv, sem):
        pltpu.async_copy(rowsv, acc_sh.at[didxv], sem, add=True)

    _zero_acc(z_hbm, acc_sh, s, rpt, tail)
    plsc.subcore_barrier()

    idx_start(0, sidxa, didxa, sia)
    idx_wait(sidxa, didxa, sia)
    gather_start(sidxa, rowsa, sga)
    idx_start(1, sidxb, didxb, sib)
    idx_wait(sidxb, didxb, sib)
    gather_start(sidxb, rowsb, sgb)

    def pair(j, _):
        i = 2 * j
        rows_wait(rowsa, sga)
        scat_start(didxa, rowsa, ssa)
        rows_wait(rowsb, sgb)
        scat_start(didxb, rowsb, ssb)
        rows_wait(rowsa, ssa)

        @pl.when(i + 2 < nch)
        def _():
            idx_start(i + 2, sidxa, didxa, sia)
            idx_wait(sidxa, didxa, sia)
            gather_start(sidxa, rowsa, sga)

        rows_wait(rowsb, ssb)

        @pl.when(i + 3 < nch)
        def _():
            idx_start(i + 3, sidxb, didxb, sib)
            idx_wait(sidxb, didxb, sib)
            gather_start(sidxb, rowsb, sgb)
        return 0
    lax.fori_loop(0, npairs, pair, 0)
    rows_wait(rowsa, sga)
    scat_start(didxa, rowsa, ssa)
    rows_wait(rowsa, ssa)
    plsc.subcore_barrier()

    _read_out(acc_sh, out_s, c, s, rpt, tail)


def _spmv(hw, src, dst, zeros_nd):
    N, D = hw.shape
    E = src.shape[0]
    nch = (E // _NW) // _CH
    assert E % (_NW * _CH) == 0 and N % _NS == 0 and nch % 2 == 1 and nch >= 3
    mesh = plsc.VectorSubcoreMesh(core_axis_name="c", subcore_axis_name="s")
    f = pl.kernel(
        functools.partial(_spmv_body, N, D, E),
        out_type=jax.ShapeDtypeStruct((_NC, N, D), _F32),
        mesh=mesh,
        scratch_types=[
            pltpu.VMEM((_CH,), jnp.int32),
            pltpu.VMEM((_CH,), jnp.int32),
            pltpu.VMEM((_CH,), jnp.int32),
            pltpu.VMEM((_CH,), jnp.int32),
            pltpu.VMEM((_CH, D), _F32),
            pltpu.VMEM((_CH, D), _F32),
            pltpu.VMEM_SHARED((N, D), _F32),
            pltpu.SemaphoreType.DMA,
            pltpu.SemaphoreType.DMA,
            pltpu.SemaphoreType.DMA,
            pltpu.SemaphoreType.DMA,
            pltpu.SemaphoreType.DMA,
            pltpu.SemaphoreType.DMA,
        ],
    )
    return f(hw, src, dst, zeros_nd)


_BN = 1000


def _tc1_body(nt_ref, w0_ref, b0_ref, w1_ref, b1_ref, ea_ref, s0_ref, out_ref):
    w0 = (jnp.dot(nt_ref[...], w0_ref[...], preferred_element_type=_F32)
          + b0_ref[...])
    h1 = jnp.maximum(
        s0_ref[0] + s0_ref[1] + ea_ref[0] + ea_ref[1] + w0, 0.0)
    out_ref[...] = (jnp.dot(h1, w1_ref[...], preferred_element_type=_F32)
                    + b1_ref[...])


def _tc1(node_table, W0, b0, W1, b1, ea_part, s0_part):
    _, N, D = ea_part.shape
    grid = (N // _BN,)
    return pl.pallas_call(
        _tc1_body,
        grid=grid,
        in_specs=[
            pl.BlockSpec((1, D), lambda i: (0, 0)),
            pl.BlockSpec((D, D), lambda i: (0, 0)),
            pl.BlockSpec((1, D), lambda i: (0, 0)),
            pl.BlockSpec((D, D), lambda i: (0, 0)),
            pl.BlockSpec((1, D), lambda i: (0, 0)),
            pl.BlockSpec((_NC, _BN, D), lambda i: (0, i, 0)),
            pl.BlockSpec((_NC, _BN, D), lambda i: (0, i, 0)),
        ],
        out_specs=pl.BlockSpec((_BN, D), lambda i: (i, 0)),
        out_shape=jax.ShapeDtypeStruct((N, D), _F32),
    )(node_table, W0, b0, W1, b1, ea_part, s0_part)


def _tc_mid_body(s_ref, ea_ref, hw_ref, w_ref, b_ref, out_ref):
    h = jnp.maximum(s_ref[0] + s_ref[1] + ea_ref[0] + ea_ref[1] + hw_ref[...],
                    0.0)
    out_ref[...] = (jnp.dot(h, w_ref[...], preferred_element_type=_F32)
                    + b_ref[...])


def _tc_mid(s_part, ea_part, hw, W, b):
    N, D = hw.shape
    grid = (N // _BN,)
    return pl.pallas_call(
        _tc_mid_body,
        grid=grid,
        in_specs=[
            pl.BlockSpec((_NC, _BN, D), lambda i: (0, i, 0)),
            pl.BlockSpec((_NC, _BN, D), lambda i: (0, i, 0)),
            pl.BlockSpec((_BN, D), lambda i: (i, 0)),
            pl.BlockSpec((D, D), lambda i: (0, 0)),
            pl.BlockSpec((1, D), lambda i: (0, 0)),
        ],
        out_specs=pl.BlockSpec((_BN, D), lambda i: (i, 0)),
        out_shape=jax.ShapeDtypeStruct((N, D), _F32),
    )(s_part, ea_part, hw, W, b)


def _tc_final_body(G, C, nblocks, s_ref, ea_ref, hw_ref, batch_ref, pw_ref,
                   pb_ref, out_ref, sums_acc, cnt_acc):
    i = pl.program_id(0)

    @pl.when(i == 0)
    def _():
        sums_acc[...] = jnp.zeros_like(sums_acc)
        cnt_acc[...] = jnp.zeros_like(cnt_acc)

    h3 = s_ref[0] + s_ref[1] + ea_ref[0] + ea_ref[1] + hw_ref[...]
    b = batch_ref[...]
    onehot = (b == lax.broadcasted_iota(jnp.int32, (_BN, G), 1)).astype(_F32)
    dn = (((0,), (0,)), ((), ()))
    sums_acc[...] += lax.dot_general(onehot, h3, dn,
                                     preferred_element_type=_F32)
    cnt_acc[...] += lax.dot_general(onehot, jnp.ones_like(h3), dn,
                                    preferred_element_type=_F32)

    @pl.when(i == nblocks - 1)
    def _():
        pooled = sums_acc[...] / jnp.maximum(cnt_acc[...], 1.0)
        out_ref[...] = (jnp.dot(pooled, pw_ref[...],
                                preferred_element_type=_F32) + pb_ref[...])


def _tc_final(s_part, ea_part, hw, batch_col, perc_W, perc_b, G):
    N, D = hw.shape
    C = perc_W.shape[1]
    nblocks = N // _BN
    return pl.pallas_call(
        functools.partial(_tc_final_body, G, C, nblocks),
        grid=(nblocks,),
        in_specs=[
            pl.BlockSpec((_NC, _BN, D), lambda i: (0, i, 0)),
            pl.BlockSpec((_NC, _BN, D), lambda i: (0, i, 0)),
            pl.BlockSpec((_BN, D), lambda i: (i, 0)),
            pl.BlockSpec((_BN, 1), lambda i: (i, 0)),
            pl.BlockSpec((D, C), lambda i: (0, 0)),
            pl.BlockSpec((1, C), lambda i: (0, 0)),
        ],
        out_specs=pl.BlockSpec((G, C), lambda i: (0, 0)),
        out_shape=jax.ShapeDtypeStruct((G, C), _F32),
        scratch_shapes=[
            pltpu.VMEM((G, D), _F32),
            pltpu.VMEM((G, D), _F32),
        ],
    )(s_part, ea_part, hw, batch_col, perc_W, perc_b)


def kernel(x, edge_index, edge_attr, batch, node_table, layer_W, layer_b,
           perc_W, perc_b):
    N = x.shape[0]
    E, D = edge_attr.shape
    L = layer_W.shape[0]
    C = perc_W.shape[1]
    G = 128

    batch_col = batch.reshape(N, 1)
    b_rows = layer_b.reshape(L, 1, D)
    pb_row = perc_b.reshape(1, C)

    w0 = node_table[0] @ layer_W[0] + layer_b[0]
    w0b = jnp.broadcast_to(w0.reshape(1, D), (_CH, D))
    zeros_nd = jnp.zeros((N, D), _F32)

    src = edge_index[0]
    dst = edge_index[1]

    ea_part, s0_part = _edge_agg_call(edge_attr, dst, w0b, zeros_nd, N)

    hw = _tc1(node_table, layer_W[0], b_rows[0], layer_W[1], b_rows[1],
              ea_part, s0_part)

    for l in range(2, L):
        s_part = _spmv(hw, src, dst, zeros_nd)
        hw = _tc_mid(s_part, ea_part, hw, layer_W[l], b_rows[l])

    s_part = _spmv(hw, src, dst, zeros_nd)
    return _tc_final(s_part, ea_part, hw, batch_col, perc_W, pb_row, G)
